# baseline (speedup 1.0000x reference)
import functools

import jax
import jax.numpy as jnp
from jax import lax
from jax.experimental import pallas as pl
from jax.experimental.pallas import tpu as pltpu
from jax.experimental.pallas import tpu_sc as plsc

N = 10000
E = 160000
G = 64
START = 5

NC = 2
NS = 16
NW = NC * NS
CH = 128
FULL = (E // NW) // CH
MAIN = NW * FULL * CH
NREM = (E - MAIN) // CH
NPAD = 10240
RPT = NPAD // NS


def _sc_mesh():
    return plsc.VectorSubcoreMesh(
        core_axis_name="c", subcore_axis_name="s", num_cores=NC, num_subcores=NS
    )


def _make_gather(D=128):
    @functools.partial(
        pl.kernel,
        out_type=jax.ShapeDtypeStruct((E, D), jnp.float32),
        mesh=_sc_mesh(),
        scratch_types=[
            pltpu.VMEM((CH,), jnp.int32),
            pltpu.VMEM((CH, D), jnp.float32),
            pltpu.SemaphoreType.DMA,
        ],
    )
    def gather(table_hbm, src_hbm, out_hbm, idx_v, rows_v, sem):
        w = lax.axis_index("s") * NC + lax.axis_index("c")
        base = w * (FULL * CH)

        def chunk(off):
            pltpu.sync_copy(src_hbm.at[pl.ds(off, CH)], idx_v)
            pltpu.async_copy(table_hbm.at[idx_v], rows_v, sem).wait()
            pltpu.sync_copy(rows_v, out_hbm.at[pl.ds(off, CH)])

        def body(j, carry):
            chunk(pl.multiple_of(base + j * CH, CH))
            return carry

        lax.fori_loop(0, FULL, body, 0)

        @pl.when(w < NREM)
        def _():
            chunk(pl.multiple_of(MAIN + w * CH, CH))

    return gather


def _make_scatter(CO=128):
    @functools.partial(
        pl.kernel,
        out_type=jax.ShapeDtypeStruct((NC, NPAD, CO), jnp.float32),
        mesh=_sc_mesh(),
        scratch_types=[
            pltpu.VMEM((CH,), jnp.int32),
            pltpu.VMEM((CH, CO), jnp.float32),
            pltpu.VMEM_SHARED((NPAD, CO), jnp.float32),
        ],
    )
    def scatter(msg_hbm, dst_hbm, zero_hbm, out_hbm, idx_v, msg_v, accu):
        c = lax.axis_index("c")
        s = lax.axis_index("s")
        w = s * NC + c
        roff = s * RPT
        pltpu.sync_copy(zero_hbm.at[pl.ds(roff, RPT)], accu.at[pl.ds(roff, RPT)])
        plsc.subcore_barrier()
        base = w * (FULL * CH)

        def chunk(off):
            pltpu.sync_copy(dst_hbm.at[pl.ds(off, CH)], idx_v)
            pltpu.sync_copy(msg_hbm.at[pl.ds(off, CH)], msg_v)
            pltpu.sync_copy(msg_v, accu.at[idx_v], add=True)

        def body(j, carry):
            chunk(pl.multiple_of(base + j * CH, CH))
            return carry

        lax.fori_loop(0, FULL, body, 0)

        @pl.when(w < NREM)
        def _():
            chunk(pl.multiple_of(MAIN + w * CH, CH))

        plsc.subcore_barrier()
        pltpu.sync_copy(accu.at[pl.ds(roff, RPT)], out_hbm.at[c, pl.ds(roff, RPT)])

    return scatter


def _msg_call(eaT, hs, w1T, b1c, w2T, b2m, CI, CO, T=128, CB=8):
    HD = w1T.shape[0]
    K2 = w2T.shape[0]

    HI = lax.Precision.HIGHEST

    def body(eaT_ref, hs_ref, w1T_ref, b1_ref, w2T_ref, b2m_ref, out_ref):
        aT = jnp.dot(w1T_ref[...], eaT_ref[...],
                     preferred_element_type=jnp.float32)
        aT = jnp.maximum(aT + b1_ref[...], 0.0)
        hsT = hs_ref[...].T
        accT = jnp.dot(b2m_ref[...], hsT[:CI, :], precision=HI,
                       preferred_element_type=jnp.float32)
        for c0 in range(0, CI, CB):
            cb = min(CB, CI - c0)
            WgT = jnp.dot(w2T_ref[c0 * CO:(c0 + cb) * CO, :], aT,
                          preferred_element_type=jnp.float32)
            for j in range(cb):
                c = c0 + j
                accT = accT + WgT[j * CO:(j + 1) * CO, :] * hsT[c:c + 1, :]
        out_ref[...] = jnp.zeros((T, 128), jnp.float32)
        out_ref[:, :CO] = accT.T

    return pl.pallas_call(
        body,
        grid=(E // T,),
        in_specs=[
            pl.BlockSpec((5, T), lambda i: (0, i)),
            pl.BlockSpec((T, hs.shape[1]), lambda i: (i, 0)),
            pl.BlockSpec((HD, 5), lambda i: (0, 0)),
            pl.BlockSpec((HD, 1), lambda i: (0, 0)),
            pl.BlockSpec((K2, HD), lambda i: (0, 0)),
            pl.BlockSpec((CO, CI), lambda i: (0, 0)),
        ],
        out_specs=pl.BlockSpec((T, 128), lambda i: (i, 0)),
        out_shape=jax.ShapeDtypeStruct((E, 128), jnp.float32),
    )(eaT, hs, w1T, b1c, w2T, b2m)


def _update_call(parts, h, root, bias, CI, CO, HOFF=0, NB=400):
    HW = h.shape[1]

    def body(p_ref, h_ref, root_ref, bias_ref, out_ref):
        hv = h_ref[...][:, HOFF:HOFF + CI]
        t = (
            p_ref[0][:, :CO]
            + p_ref[1][:, :CO]
            + jnp.dot(hv, root_ref[...], preferred_element_type=jnp.float32)
            + bias_ref[...][None, :]
        )
        out_ref[...] = jnp.zeros((NB, 128), jnp.float32)
        out_ref[:, :CO] = jnp.where(t > 0, t, jnp.exp(jnp.minimum(t, 0.0)) - 1.0)

    return pl.pallas_call(
        body,
        grid=(N // NB,),
        in_specs=[
            pl.BlockSpec((NC, NB, 128), lambda i: (0, i, 0)),
            pl.BlockSpec((NB, HW), lambda i: (i, 0)),
            pl.BlockSpec((CI, CO), lambda i: (0, 0)),
            pl.BlockSpec((CO,), lambda i: (0,)),
        ],
        out_specs=pl.BlockSpec((NB, 128), lambda i: (i, 0)),
        out_shape=jax.ShapeDtypeStruct((N, 128), jnp.float32),
    )(parts, h, root, bias)


def _pool_sums_call(h, xp, batch3, NB=400):
    def body(h_ref, xp_ref, b_ref, sh_ref, sxc_ref):
        i = pl.program_id(0)

        @pl.when(i == 0)
        def _():
            sh_ref[...] = jnp.zeros_like(sh_ref)
            sxc_ref[...] = jnp.zeros_like(sxc_ref)

        bat = b_ref[0, 0, :]
        gids = lax.broadcasted_iota(jnp.int32, (G, NB), 0)
        oh = (bat[None, :] == gids).astype(jnp.float32)
        sh_ref[...] += jnp.dot(oh, h_ref[...][:, :64], precision=lax.Precision.HIGHEST,
                               preferred_element_type=jnp.float32)
        sxc_ref[...] += jnp.dot(oh, xp_ref[...], precision=lax.Precision.HIGHEST,
                                preferred_element_type=jnp.float32)

    return pl.pallas_call(
        body,
        grid=(N // NB,),
        in_specs=[
            pl.BlockSpec((NB, h.shape[1]), lambda i: (i, 0)),
            pl.BlockSpec((NB, 17), lambda i: (i, 0)),
            pl.BlockSpec((1, 1, NB), lambda i: (i, 0, 0)),
        ],
        out_specs=[
            pl.BlockSpec((G, 64), lambda i: (0, 0)),
            pl.BlockSpec((G, 17), lambda i: (0, 0)),
        ],
        out_shape=[
            jax.ShapeDtypeStruct((G, 64), jnp.float32),
            jax.ShapeDtypeStruct((G, 17), jnp.float32),
        ],
    )(h, xp, batch3)


def _head_call(sh, sxc, w_top, w_xpad, fc1_b, fc2_w, fc2_b, fc3_w, fc3_b):
    def body(sh_ref, sxc_ref, wt_ref, wx_ref, b1_ref, w2_ref, b2_ref, w3_ref,
             b3_ref, out_ref):
        sxcv = sxc_ref[...]
        cnt = sxcv[:, 16][:, None]
        inv = 1.0 / jnp.maximum(cnt, 1.0)
        mh = sh_ref[...] * inv
        mx = sxcv * inv
        o = (
            jnp.dot(mh, wt_ref[...], preferred_element_type=jnp.float32)
            + jnp.dot(mx, wx_ref[...], preferred_element_type=jnp.float32)
            + b1_ref[...][None, :]
        )
        o = jnp.where(o > 0, o, jnp.exp(jnp.minimum(o, 0.0)) - 1.0)
        o = jnp.dot(o, w2_ref[...], preferred_element_type=jnp.float32) + b2_ref[...][None, :]
        o = jnp.where(o > 0, o, jnp.exp(jnp.minimum(o, 0.0)) - 1.0)
        o = jnp.dot(o, w3_ref[...], preferred_element_type=jnp.float32) + b3_ref[...][None, :]
        out_ref[...] = o

    return pl.pallas_call(
        body,
        out_shape=jax.ShapeDtypeStruct((G, 1), jnp.float32),
    )(sh, sxc, w_top, w_xpad, fc1_b, fc2_w, fc2_b, fc3_w, fc3_b)


def kernel(x, edge_index, edge_attr, batch, c1_w1, c1_b1, c1_w2, c1_b2, c1_root, c1_bias, c2_w1, c2_b1, c2_w2, c2_b2, c2_root, c2_bias, c3_w1, c3_b1, c3_w2, c3_b2, c3_root, c3_bias, fc1_w, fc1_b, fc2_w, fc2_b, fc3_w, fc3_b):
    src = edge_index[0]
    dst = edge_index[1]
    zeros128 = jnp.zeros((NPAD, 128), jnp.float32)
    eaT = edge_attr.T

    def tw(w1, b1, w2, b2, CI, CO):
        return (w1.T, b1[:, None], w2.T, b2.reshape(CI, CO).T)

    xpad = jnp.zeros((N, 128), jnp.float32).at[:, :16].set(x)
    hs0 = _make_gather()(xpad, src)
    msg1 = _msg_call(eaT, hs0, *tw(c1_w1, c1_b1, c1_w2, c1_b2, 5, 32),
                     CI=5, CO=32, CB=5)
    p1 = _make_scatter()(msg1, dst, zeros128)
    h1 = _update_call(p1, x, c1_root, c1_bias, CI=5, CO=32, HOFF=0)

    hs1 = _make_gather()(h1, src)
    msg2 = _msg_call(eaT, hs1, *tw(c2_w1, c2_b1, c2_w2, c2_b2, 32, 64),
                     CI=32, CO=64)
    p2 = _make_scatter()(msg2, dst, zeros128)
    h2 = _update_call(p2, h1, c2_root, c2_bias, CI=32, CO=64)

    hs2 = _make_gather()(h2, src)
    msg3 = _msg_call(eaT, hs2, *tw(c3_w1, c3_b1, c3_w2, c3_b2, 64, 64),
                     CI=64, CO=64)
    p3 = _make_scatter()(msg3, dst, zeros128)
    h3 = _update_call(p3, h2, c3_root, c3_bias, CI=64, CO=64)

    xp = jnp.concatenate([x, jnp.ones((N, 1), jnp.float32)], axis=1)
    batch3 = batch.reshape(N // 400, 1, 400)
    sh, sxc = _pool_sums_call(h3, xp, batch3)
    w_xpad = jnp.zeros((17, 32), jnp.float32).at[START:16].set(fc1_w[64:])
    out = _head_call(sh, sxc, fc1_w[:64], w_xpad, fc1_b, fc2_w, fc2_b, fc3_w,
                     fc3_b)
    return out.reshape(-1)


# speedup vs baseline: 1.9244x; 1.9244x over previous
import functools

import jax
import jax.numpy as jnp
from jax import lax
from jax.experimental import pallas as pl
from jax.experimental.pallas import tpu as pltpu
from jax.experimental.pallas import tpu_sc as plsc

N = 10000
E = 160000
G = 64
START = 5

NC = 2
NS = 16
NW = NC * NS
CH = 128
FULL = (E // NW) // CH
MAIN = NW * FULL * CH
NREM = (E - MAIN) // CH
NPAD = 10240
RPT = NPAD // NS


def _sc_mesh():
    return plsc.VectorSubcoreMesh(
        core_axis_name="c", subcore_axis_name="s", num_cores=NC, num_subcores=NS
    )


def _make_gather(D=128):
    @functools.partial(
        pl.kernel,
        out_type=jax.ShapeDtypeStruct((E, D), jnp.float32),
        mesh=_sc_mesh(),
        scratch_types=[
            pltpu.VMEM((CH,), jnp.int32),
            pltpu.VMEM((CH, D), jnp.float32),
            pltpu.SemaphoreType.DMA,
        ],
    )
    def gather(table_hbm, src_hbm, out_hbm, idx_v, rows_v, sem):
        w = lax.axis_index("s") * NC + lax.axis_index("c")
        base = w * (FULL * CH)

        def chunk(off):
            pltpu.sync_copy(src_hbm.at[pl.ds(off, CH)], idx_v)
            pltpu.async_copy(table_hbm.at[idx_v], rows_v, sem).wait()
            pltpu.sync_copy(rows_v, out_hbm.at[pl.ds(off, CH)])

        def body(j, carry):
            chunk(pl.multiple_of(base + j * CH, CH))
            return carry

        lax.fori_loop(0, FULL, body, 0)

        @pl.when(w < NREM)
        def _():
            chunk(pl.multiple_of(MAIN + w * CH, CH))

    return gather


def _make_scatter(CO=128):
    @functools.partial(
        pl.kernel,
        out_type=jax.ShapeDtypeStruct((NC, NPAD, CO), jnp.float32),
        mesh=_sc_mesh(),
        scratch_types=[
            pltpu.VMEM((CH,), jnp.int32),
            pltpu.VMEM((CH, CO), jnp.float32),
            pltpu.VMEM_SHARED((NPAD, CO), jnp.float32),
        ],
    )
    def scatter(msg_hbm, dst_hbm, zero_hbm, out_hbm, idx_v, msg_v, accu):
        c = lax.axis_index("c")
        s = lax.axis_index("s")
        w = s * NC + c
        roff = s * RPT
        pltpu.sync_copy(zero_hbm.at[pl.ds(roff, RPT)], accu.at[pl.ds(roff, RPT)])
        plsc.subcore_barrier()
        base = w * (FULL * CH)

        def chunk(off):
            pltpu.sync_copy(dst_hbm.at[pl.ds(off, CH)], idx_v)
            pltpu.sync_copy(msg_hbm.at[pl.ds(off, CH)], msg_v)
            pltpu.sync_copy(msg_v, accu.at[idx_v], add=True)

        def body(j, carry):
            chunk(pl.multiple_of(base + j * CH, CH))
            return carry

        lax.fori_loop(0, FULL, body, 0)

        @pl.when(w < NREM)
        def _():
            chunk(pl.multiple_of(MAIN + w * CH, CH))

        plsc.subcore_barrier()
        pltpu.sync_copy(accu.at[pl.ds(roff, RPT)], out_hbm.at[c, pl.ds(roff, RPT)])

    return scatter


def _msg_call(eaT, hs, w1T, b1c, w2T, b2m, CI, CO, T=640, CB=8):
    HD = w1T.shape[0]
    K2 = w2T.shape[0]

    HI = lax.Precision.HIGHEST

    def body(eaT_ref, hs_ref, w1T_ref, b1_ref, w2T_ref, b2m_ref, out_ref):
        aT = jnp.dot(w1T_ref[...], eaT_ref[...],
                     preferred_element_type=jnp.float32)
        aT = jnp.maximum(aT + b1_ref[...], 0.0)
        hsT = hs_ref[...].T
        accT = jnp.dot(b2m_ref[...], hsT[:CI, :], precision=HI,
                       preferred_element_type=jnp.float32)
        for c0 in range(0, CI, CB):
            cb = min(CB, CI - c0)
            WgT = jnp.dot(w2T_ref[c0 * CO:(c0 + cb) * CO, :], aT,
                          preferred_element_type=jnp.float32)
            for j in range(cb):
                c = c0 + j
                accT = accT + WgT[j * CO:(j + 1) * CO, :] * hsT[c:c + 1, :]
        out_ref[...] = jnp.zeros((T, 128), jnp.float32)
        out_ref[:, :CO] = accT.T

    return pl.pallas_call(
        body,
        grid=(E // T,),
        in_specs=[
            pl.BlockSpec((5, T), lambda i: (0, i)),
            pl.BlockSpec((T, hs.shape[1]), lambda i: (i, 0)),
            pl.BlockSpec((HD, 5), lambda i: (0, 0)),
            pl.BlockSpec((HD, 1), lambda i: (0, 0)),
            pl.BlockSpec((K2, HD), lambda i: (0, 0)),
            pl.BlockSpec((CO, CI), lambda i: (0, 0)),
        ],
        out_specs=pl.BlockSpec((T, 128), lambda i: (i, 0)),
        out_shape=jax.ShapeDtypeStruct((E, 128), jnp.float32),
    )(eaT, hs, w1T, b1c, w2T, b2m)


def _update_call(parts, h, root, bias, CI, CO, HOFF=0, NB=400):
    HW = h.shape[1]

    def body(p_ref, h_ref, root_ref, bias_ref, out_ref):
        hv = h_ref[...][:, HOFF:HOFF + CI]
        t = (
            p_ref[0][:, :CO]
            + p_ref[1][:, :CO]
            + jnp.dot(hv, root_ref[...], preferred_element_type=jnp.float32)
            + bias_ref[...][None, :]
        )
        out_ref[...] = jnp.zeros((NB, 128), jnp.float32)
        out_ref[:, :CO] = jnp.where(t > 0, t, jnp.exp(jnp.minimum(t, 0.0)) - 1.0)

    return pl.pallas_call(
        body,
        grid=(N // NB,),
        in_specs=[
            pl.BlockSpec((NC, NB, 128), lambda i: (0, i, 0)),
            pl.BlockSpec((NB, HW), lambda i: (i, 0)),
            pl.BlockSpec((CI, CO), lambda i: (0, 0)),
            pl.BlockSpec((CO,), lambda i: (0,)),
        ],
        out_specs=pl.BlockSpec((NB, 128), lambda i: (i, 0)),
        out_shape=jax.ShapeDtypeStruct((N, 128), jnp.float32),
    )(parts, h, root, bias)


def _pool_sums_call(h, xp, batch3, NB=400):
    def body(h_ref, xp_ref, b_ref, sh_ref, sxc_ref):
        i = pl.program_id(0)

        @pl.when(i == 0)
        def _():
            sh_ref[...] = jnp.zeros_like(sh_ref)
            sxc_ref[...] = jnp.zeros_like(sxc_ref)

        bat = b_ref[0, 0, :]
        gids = lax.broadcasted_iota(jnp.int32, (G, NB), 0)
        oh = (bat[None, :] == gids).astype(jnp.float32)
        sh_ref[...] += jnp.dot(oh, h_ref[...][:, :64], precision=lax.Precision.HIGHEST,
                               preferred_element_type=jnp.float32)
        sxc_ref[...] += jnp.dot(oh, xp_ref[...], precision=lax.Precision.HIGHEST,
                                preferred_element_type=jnp.float32)

    return pl.pallas_call(
        body,
        grid=(N // NB,),
        in_specs=[
            pl.BlockSpec((NB, h.shape[1]), lambda i: (i, 0)),
            pl.BlockSpec((NB, 17), lambda i: (i, 0)),
            pl.BlockSpec((1, 1, NB), lambda i: (i, 0, 0)),
        ],
        out_specs=[
            pl.BlockSpec((G, 64), lambda i: (0, 0)),
            pl.BlockSpec((G, 17), lambda i: (0, 0)),
        ],
        out_shape=[
            jax.ShapeDtypeStruct((G, 64), jnp.float32),
            jax.ShapeDtypeStruct((G, 17), jnp.float32),
        ],
    )(h, xp, batch3)


def _head_call(sh, sxc, w_top, w_xpad, fc1_b, fc2_w, fc2_b, fc3_w, fc3_b):
    def body(sh_ref, sxc_ref, wt_ref, wx_ref, b1_ref, w2_ref, b2_ref, w3_ref,
             b3_ref, out_ref):
        sxcv = sxc_ref[...]
        cnt = sxcv[:, 16][:, None]
        inv = 1.0 / jnp.maximum(cnt, 1.0)
        mh = sh_ref[...] * inv
        mx = sxcv * inv
        o = (
            jnp.dot(mh, wt_ref[...], preferred_element_type=jnp.float32)
            + jnp.dot(mx, wx_ref[...], preferred_element_type=jnp.float32)
            + b1_ref[...][None, :]
        )
        o = jnp.where(o > 0, o, jnp.exp(jnp.minimum(o, 0.0)) - 1.0)
        o = jnp.dot(o, w2_ref[...], preferred_element_type=jnp.float32) + b2_ref[...][None, :]
        o = jnp.where(o > 0, o, jnp.exp(jnp.minimum(o, 0.0)) - 1.0)
        o = jnp.dot(o, w3_ref[...], preferred_element_type=jnp.float32) + b3_ref[...][None, :]
        out_ref[...] = o

    return pl.pallas_call(
        body,
        out_shape=jax.ShapeDtypeStruct((G, 1), jnp.float32),
    )(sh, sxc, w_top, w_xpad, fc1_b, fc2_w, fc2_b, fc3_w, fc3_b)


def kernel(x, edge_index, edge_attr, batch, c1_w1, c1_b1, c1_w2, c1_b2, c1_root, c1_bias, c2_w1, c2_b1, c2_w2, c2_b2, c2_root, c2_bias, c3_w1, c3_b1, c3_w2, c3_b2, c3_root, c3_bias, fc1_w, fc1_b, fc2_w, fc2_b, fc3_w, fc3_b):
    src = edge_index[0]
    dst = edge_index[1]
    zeros128 = jnp.zeros((NPAD, 128), jnp.float32)
    eaT = edge_attr.T

    def tw(w1, b1, w2, b2, CI, CO):
        return (w1.T, b1[:, None], w2.T, b2.reshape(CI, CO).T)

    xpad = jnp.zeros((N, 128), jnp.float32).at[:, :16].set(x)
    hs0 = _make_gather()(xpad, src)
    msg1 = _msg_call(eaT, hs0, *tw(c1_w1, c1_b1, c1_w2, c1_b2, 5, 32),
                     CI=5, CO=32, CB=5)
    p1 = _make_scatter()(msg1, dst, zeros128)
    h1 = _update_call(p1, x, c1_root, c1_bias, CI=5, CO=32, HOFF=0)

    hs1 = _make_gather()(h1, src)
    msg2 = _msg_call(eaT, hs1, *tw(c2_w1, c2_b1, c2_w2, c2_b2, 32, 64),
                     CI=32, CO=64)
    p2 = _make_scatter()(msg2, dst, zeros128)
    h2 = _update_call(p2, h1, c2_root, c2_bias, CI=32, CO=64)

    hs2 = _make_gather()(h2, src)
    msg3 = _msg_call(eaT, hs2, *tw(c3_w1, c3_b1, c3_w2, c3_b2, 64, 64),
                     CI=64, CO=64)
    p3 = _make_scatter()(msg3, dst, zeros128)
    h3 = _update_call(p3, h2, c3_root, c3_bias, CI=64, CO=64)

    xp = jnp.concatenate([x, jnp.ones((N, 1), jnp.float32)], axis=1)
    batch3 = batch.reshape(N // 400, 1, 400)
    sh, sxc = _pool_sums_call(h3, xp, batch3)
    w_xpad = jnp.zeros((17, 32), jnp.float32).at[START:16].set(fc1_w[64:])
    out = _head_call(sh, sxc, fc1_w[:64], w_xpad, fc1_b, fc2_w, fc2_b, fc3_w,
                     fc3_b)
    return out.reshape(-1)


# speedup vs baseline: 2.1597x; 1.1222x over previous
import functools

import jax
import jax.numpy as jnp
from jax import lax
from jax.experimental import pallas as pl
from jax.experimental.pallas import tpu as pltpu
from jax.experimental.pallas import tpu_sc as plsc

N = 10000
E = 160000
G = 64
START = 5

NC = 2
NS = 16
NW = NC * NS
CH = 128
FULL = (E // NW) // CH
MAIN = NW * FULL * CH
NREM = (E - MAIN) // CH
NPAD = 10240
RPT = NPAD // NS


def _sc_mesh():
    return plsc.VectorSubcoreMesh(
        core_axis_name="c", subcore_axis_name="s", num_cores=NC, num_subcores=NS
    )


def _make_gather(D=128):
    @functools.partial(
        pl.kernel,
        out_type=jax.ShapeDtypeStruct((E, D), jnp.float32),
        mesh=_sc_mesh(),
        scratch_types=[
            pltpu.VMEM((CH,), jnp.int32),
            pltpu.VMEM((CH, D), jnp.float32),
            pltpu.SemaphoreType.DMA,
        ],
    )
    def gather(table_hbm, src_hbm, out_hbm, idx_v, rows_v, sem):
        w = lax.axis_index("s") * NC + lax.axis_index("c")
        base = w * (FULL * CH)

        def chunk(off):
            pltpu.sync_copy(src_hbm.at[pl.ds(off, CH)], idx_v)
            pltpu.async_copy(table_hbm.at[idx_v], rows_v, sem).wait()
            pltpu.sync_copy(rows_v, out_hbm.at[pl.ds(off, CH)])

        def body(j, carry):
            chunk(pl.multiple_of(base + j * CH, CH))
            return carry

        lax.fori_loop(0, FULL, body, 0)

        @pl.when(w < NREM)
        def _():
            chunk(pl.multiple_of(MAIN + w * CH, CH))

    return gather


def _make_scatter(CO=128):
    @functools.partial(
        pl.kernel,
        out_type=jax.ShapeDtypeStruct((NC, NPAD, CO), jnp.float32),
        mesh=_sc_mesh(),
        scratch_types=[
            pltpu.VMEM((CH,), jnp.int32),
            pltpu.VMEM((CH, CO), jnp.float32),
            pltpu.VMEM_SHARED((NPAD, CO), jnp.float32),
        ],
    )
    def scatter(msg_hbm, dst_hbm, zero_hbm, out_hbm, idx_v, msg_v, accu):
        c = lax.axis_index("c")
        s = lax.axis_index("s")
        w = s * NC + c
        roff = s * RPT
        pltpu.sync_copy(zero_hbm.at[pl.ds(roff, RPT)], accu.at[pl.ds(roff, RPT)])
        plsc.subcore_barrier()
        base = w * (FULL * CH)

        def chunk(off):
            pltpu.sync_copy(dst_hbm.at[pl.ds(off, CH)], idx_v)
            pltpu.sync_copy(msg_hbm.at[pl.ds(off, CH)], msg_v)
            pltpu.sync_copy(msg_v, accu.at[idx_v], add=True)

        def body(j, carry):
            chunk(pl.multiple_of(base + j * CH, CH))
            return carry

        lax.fori_loop(0, FULL, body, 0)

        @pl.when(w < NREM)
        def _():
            chunk(pl.multiple_of(MAIN + w * CH, CH))

        plsc.subcore_barrier()
        pltpu.sync_copy(accu.at[pl.ds(roff, RPT)], out_hbm.at[c, pl.ds(roff, RPT)])

    return scatter


def _msg_call(eaT, hs, w1T, b1c, w2T, b2m, CI, CO, T=1280, CB=8):
    HD = w1T.shape[0]
    K2 = w2T.shape[0]

    HI = lax.Precision.HIGHEST

    def body(eaT_ref, hs_ref, w1T_ref, b1_ref, w2T_ref, b2m_ref, out_ref):
        aT = jnp.dot(w1T_ref[...], eaT_ref[...],
                     preferred_element_type=jnp.float32)
        aT = jnp.maximum(aT + b1_ref[...], 0.0)
        hsT = hs_ref[...].T
        accT = jnp.dot(b2m_ref[...], hsT[:CI, :], precision=HI,
                       preferred_element_type=jnp.float32)
        for c0 in range(0, CI, CB):
            cb = min(CB, CI - c0)
            WgT = jnp.dot(w2T_ref[c0 * CO:(c0 + cb) * CO, :], aT,
                          preferred_element_type=jnp.float32)
            for j in range(cb):
                c = c0 + j
                accT = accT + WgT[j * CO:(j + 1) * CO, :] * hsT[c:c + 1, :]
        out_ref[...] = jnp.zeros((T, 128), jnp.float32)
        out_ref[:, :CO] = accT.T

    return pl.pallas_call(
        body,
        grid=(E // T,),
        in_specs=[
            pl.BlockSpec((5, T), lambda i: (0, i)),
            pl.BlockSpec((T, hs.shape[1]), lambda i: (i, 0)),
            pl.BlockSpec((HD, 5), lambda i: (0, 0)),
            pl.BlockSpec((HD, 1), lambda i: (0, 0)),
            pl.BlockSpec((K2, HD), lambda i: (0, 0)),
            pl.BlockSpec((CO, CI), lambda i: (0, 0)),
        ],
        out_specs=pl.BlockSpec((T, 128), lambda i: (i, 0)),
        out_shape=jax.ShapeDtypeStruct((E, 128), jnp.float32),
    )(eaT, hs, w1T, b1c, w2T, b2m)


def _update_call(parts, h, root, bias, CI, CO, HOFF=0, NB=400):
    HW = h.shape[1]

    def body(p_ref, h_ref, root_ref, bias_ref, out_ref):
        hv = h_ref[...][:, HOFF:HOFF + CI]
        t = (
            p_ref[0][:, :CO]
            + p_ref[1][:, :CO]
            + jnp.dot(hv, root_ref[...], preferred_element_type=jnp.float32)
            + bias_ref[...][None, :]
        )
        out_ref[...] = jnp.zeros((NB, 128), jnp.float32)
        out_ref[:, :CO] = jnp.where(t > 0, t, jnp.exp(jnp.minimum(t, 0.0)) - 1.0)

    return pl.pallas_call(
        body,
        grid=(N // NB,),
        in_specs=[
            pl.BlockSpec((NC, NB, 128), lambda i: (0, i, 0)),
            pl.BlockSpec((NB, HW), lambda i: (i, 0)),
            pl.BlockSpec((CI, CO), lambda i: (0, 0)),
            pl.BlockSpec((CO,), lambda i: (0,)),
        ],
        out_specs=pl.BlockSpec((NB, 128), lambda i: (i, 0)),
        out_shape=jax.ShapeDtypeStruct((N, 128), jnp.float32),
    )(parts, h, root, bias)


def _pool_sums_call(h, xp, batch3, NB=400):
    def body(h_ref, xp_ref, b_ref, sh_ref, sxc_ref):
        i = pl.program_id(0)

        @pl.when(i == 0)
        def _():
            sh_ref[...] = jnp.zeros_like(sh_ref)
            sxc_ref[...] = jnp.zeros_like(sxc_ref)

        bat = b_ref[0, 0, :]
        gids = lax.broadcasted_iota(jnp.int32, (G, NB), 0)
        oh = (bat[None, :] == gids).astype(jnp.float32)
        sh_ref[...] += jnp.dot(oh, h_ref[...][:, :64], precision=lax.Precision.HIGHEST,
                               preferred_element_type=jnp.float32)
        sxc_ref[...] += jnp.dot(oh, xp_ref[...], precision=lax.Precision.HIGHEST,
                                preferred_element_type=jnp.float32)

    return pl.pallas_call(
        body,
        grid=(N // NB,),
        in_specs=[
            pl.BlockSpec((NB, h.shape[1]), lambda i: (i, 0)),
            pl.BlockSpec((NB, 17), lambda i: (i, 0)),
            pl.BlockSpec((1, 1, NB), lambda i: (i, 0, 0)),
        ],
        out_specs=[
            pl.BlockSpec((G, 64), lambda i: (0, 0)),
            pl.BlockSpec((G, 17), lambda i: (0, 0)),
        ],
        out_shape=[
            jax.ShapeDtypeStruct((G, 64), jnp.float32),
            jax.ShapeDtypeStruct((G, 17), jnp.float32),
        ],
    )(h, xp, batch3)


def _head_call(sh, sxc, w_top, w_xpad, fc1_b, fc2_w, fc2_b, fc3_w, fc3_b):
    def body(sh_ref, sxc_ref, wt_ref, wx_ref, b1_ref, w2_ref, b2_ref, w3_ref,
             b3_ref, out_ref):
        sxcv = sxc_ref[...]
        cnt = sxcv[:, 16][:, None]
        inv = 1.0 / jnp.maximum(cnt, 1.0)
        mh = sh_ref[...] * inv
        mx = sxcv * inv
        o = (
            jnp.dot(mh, wt_ref[...], preferred_element_type=jnp.float32)
            + jnp.dot(mx, wx_ref[...], preferred_element_type=jnp.float32)
            + b1_ref[...][None, :]
        )
        o = jnp.where(o > 0, o, jnp.exp(jnp.minimum(o, 0.0)) - 1.0)
        o = jnp.dot(o, w2_ref[...], preferred_element_type=jnp.float32) + b2_ref[...][None, :]
        o = jnp.where(o > 0, o, jnp.exp(jnp.minimum(o, 0.0)) - 1.0)
        o = jnp.dot(o, w3_ref[...], preferred_element_type=jnp.float32) + b3_ref[...][None, :]
        out_ref[...] = o

    return pl.pallas_call(
        body,
        out_shape=jax.ShapeDtypeStruct((G, 1), jnp.float32),
    )(sh, sxc, w_top, w_xpad, fc1_b, fc2_w, fc2_b, fc3_w, fc3_b)


def kernel(x, edge_index, edge_attr, batch, c1_w1, c1_b1, c1_w2, c1_b2, c1_root, c1_bias, c2_w1, c2_b1, c2_w2, c2_b2, c2_root, c2_bias, c3_w1, c3_b1, c3_w2, c3_b2, c3_root, c3_bias, fc1_w, fc1_b, fc2_w, fc2_b, fc3_w, fc3_b):
    src = edge_index[0]
    dst = edge_index[1]
    zeros128 = jnp.zeros((NPAD, 128), jnp.float32)
    eaT = edge_attr.T

    def tw(w1, b1, w2, b2, CI, CO):
        return (w1.T, b1[:, None], w2.T, b2.reshape(CI, CO).T)

    xpad = jnp.zeros((N, 128), jnp.float32).at[:, :16].set(x)
    hs0 = _make_gather()(xpad, src)
    msg1 = _msg_call(eaT, hs0, *tw(c1_w1, c1_b1, c1_w2, c1_b2, 5, 32),
                     CI=5, CO=32, CB=5)
    p1 = _make_scatter()(msg1, dst, zeros128)
    h1 = _update_call(p1, x, c1_root, c1_bias, CI=5, CO=32, HOFF=0)

    hs1 = _make_gather()(h1, src)
    msg2 = _msg_call(eaT, hs1, *tw(c2_w1, c2_b1, c2_w2, c2_b2, 32, 64),
                     CI=32, CO=64)
    p2 = _make_scatter()(msg2, dst, zeros128)
    h2 = _update_call(p2, h1, c2_root, c2_bias, CI=32, CO=64)

    hs2 = _make_gather()(h2, src)
    msg3 = _msg_call(eaT, hs2, *tw(c3_w1, c3_b1, c3_w2, c3_b2, 64, 64),
                     CI=64, CO=64)
    p3 = _make_scatter()(msg3, dst, zeros128)
    h3 = _update_call(p3, h2, c3_root, c3_bias, CI=64, CO=64)

    xp = jnp.concatenate([x, jnp.ones((N, 1), jnp.float32)], axis=1)
    batch3 = batch.reshape(N // 400, 1, 400)
    sh, sxc = _pool_sums_call(h3, xp, batch3)
    w_xpad = jnp.zeros((17, 32), jnp.float32).at[START:16].set(fc1_w[64:])
    out = _head_call(sh, sxc, fc1_w[:64], w_xpad, fc1_b, fc2_w, fc2_b, fc3_w,
                     fc3_b)
    return out.reshape(-1)
